# baseline (speedup 1.0000x reference)
import jax
import jax.numpy as jnp
from jax import lax
from jax.experimental import pallas as pl
from jax.experimental.pallas import tpu as pltpu
from jax.experimental.pallas import tpu_sc as plsc

NUM_EMB = 1000000
G_DIM = 64
R_DIM = 64
OUT_DIM = G_DIM + R_DIM
BATCH = 4096
NB_WORDS = 50
B_TOTAL = BATCH * NB_WORDS

NC = 2
NS = 16
NW = NC * NS
B_PER_W = B_TOTAL // NW
GROUP = 128
N_GROUPS = B_PER_W // GROUP

JB = 128
JB_SHIFT = 7
J0_MAX = 1000064 - JB
NSLOT = 4


def _emb_body(sidx_hbm, spos_hbm, g_hbm, r_hbm, out_hbm,
              sidx_v, spos_v, blk, comb, bsem, ssem):
    wid = lax.axis_index("s") * NC + lax.axis_index("c")
    lane = lax.iota(jnp.int32, 16)

    def window(b):
        return jnp.minimum(b * JB, J0_MAX)

    def fetch_block(b, wait):
        m = b & (NSLOT - 1)
        j0 = window(b)
        cg = pltpu.async_copy(g_hbm.at[:, pl.ds(j0, JB)],
                              blk.at[m, pl.ds(0, G_DIM), :], bsem.at[m])
        cr = pltpu.async_copy(r_hbm.at[:, pl.ds(j0, JB)],
                              blk.at[m, pl.ds(G_DIM, R_DIM), :], bsem.at[m])
        if wait:
            cg.wait()
            cr.wait()

    def drain_block(b):
        m = b & (NSLOT - 1)
        pltpu.make_async_copy(
            g_hbm.at[:, pl.ds(0, JB)], blk.at[m], bsem.at[m]).wait()

    def drain_scatter(g):
        p = g & 1
        pltpu.make_async_copy(
            out_hbm.at[pl.ds(0, GROUP), :], comb.at[p], ssem.at[p]).wait()

    pltpu.sync_copy(sidx_hbm.at[wid], sidx_v.at[pl.ds(0, B_PER_W)])
    pltpu.sync_copy(spos_hbm.at[wid], spos_v)

    b0 = sidx_v[pl.ds(0, 16)][0] >> JB_SHIFT
    fetch_block(b0, True)
    fetch_block(b0 + 1, False)
    fetch_block(b0 + 2, False)
    fetch_block(b0 + 3, False)

    def hit_body(sl, cur_b):
        j = sidx_v[pl.ds(sl, 16)][0]
        b = j >> JB_SHIFT
        reload = b != cur_b
        skip = b != cur_b + 1

        @pl.when(reload & jnp.logical_not(skip))
        def _():
            drain_block(b)
            fetch_block(b + NSLOT - 1, False)

        @pl.when(reload & skip)
        def _():
            drain_block(cur_b + 1)
            drain_block(cur_b + 2)
            drain_block(cur_b + 3)
            fetch_block(b, True)
            fetch_block(b + 1, False)
            fetch_block(b + 2, False)
            fetch_block(b + 3, False)

        g = sl >> 7
        rowi = sl & (GROUP - 1)

        @pl.when((rowi == 0) & (g >= 2))
        def _():
            drain_scatter(g)

        m = b & (NSLOT - 1)
        colv = jnp.full((16,), j - window(b), jnp.int32)
        mv = jnp.full((16,), m, jnp.int32)
        gp = g & 1
        for gi in range(0, G_DIM, 16):
            comb[gp, rowi, pl.ds(gi, 16)] = plsc.load_gather(
                blk, [mv, lane + gi, colv])
        for gi in range(0, R_DIM, 16):
            comb[gp, rowi, pl.ds(G_DIM + gi, 16)] = plsc.load_gather(
                blk, [mv, G_DIM + lane + gi, colv])

        @pl.when(rowi == GROUP - 1)
        def _():
            pltpu.async_copy(
                comb.at[gp], out_hbm.at[spos_v.at[g]], ssem.at[gp])

        return b

    last_b = lax.fori_loop(0, B_PER_W, hit_body, b0)

    drain_block(last_b + 1)
    drain_block(last_b + 2)
    drain_block(last_b + 3)
    drain_scatter(N_GROUPS - 2)
    drain_scatter(N_GROUPS - 1)


def _emb_call(sidx2, spos3, glove_t, rand_t):
    kern = pl.kernel(
        _emb_body,
        out_type=jax.ShapeDtypeStruct((B_TOTAL, OUT_DIM), jnp.float32),
        mesh=plsc.VectorSubcoreMesh(core_axis_name="c", subcore_axis_name="s"),
        compiler_params=pltpu.CompilerParams(needs_layout_passes=False),
        scratch_types=[
            pltpu.VMEM((B_PER_W + 16,), jnp.int32),
            pltpu.VMEM((N_GROUPS, GROUP), jnp.int32),
            pltpu.VMEM((NSLOT, OUT_DIM, JB), jnp.float32),
            pltpu.VMEM((2, GROUP, OUT_DIM), jnp.float32),
            pltpu.SemaphoreType.DMA((NSLOT,)),
            pltpu.SemaphoreType.DMA((2,)),
        ],
    )
    return kern(sidx2, spos3, glove_t, rand_t)


def kernel(x, glove_weight, rand_weight):
    x_flat = x.reshape(B_TOTAL)
    pos = lax.iota(jnp.int32, B_TOTAL)
    sidx, spos = lax.sort([x_flat, pos], num_keys=1)
    sidx2 = sidx.reshape(NW, B_PER_W)
    spos3 = spos.reshape(NW, N_GROUPS, GROUP)
    out = _emb_call(sidx2, spos3, glove_weight.T, rand_weight.T)
    return out.reshape(BATCH, NB_WORDS, OUT_DIM)


# speedup vs baseline: 1.1532x; 1.1532x over previous
import jax
import jax.numpy as jnp
from jax import lax
from jax.experimental import pallas as pl
from jax.experimental.pallas import tpu as pltpu
from jax.experimental.pallas import tpu_sc as plsc

NUM_EMB = 1000000
G_DIM = 64
R_DIM = 64
OUT_DIM = G_DIM + R_DIM
BATCH = 4096
NB_WORDS = 50
B_TOTAL = BATCH * NB_WORDS

NC = 2
NS = 16
NW = NC * NS
B_PER_W = B_TOTAL // NW
GROUP = 64
N_GROUPS = B_PER_W // GROUP
N_CHUNKS = B_PER_W // 16

JB = 256
JB_SHIFT = 8
J0_MAX = 1000064 - JB
NSLOT = 2


def _emb_body(sidx_hbm, spos_hbm, g_hbm, r_hbm, out_hbm,
              sidx_v, spos_v, blk, comb, colsc, bsem, ssem):
    wid = lax.axis_index("s") * NC + lax.axis_index("c")
    lane = lax.iota(jnp.int32, 16)
    hsplat = [jnp.full((16,), h, jnp.int32) for h in range(16)]

    def window(b):
        return jnp.minimum(b * JB, J0_MAX)

    def slot(b):
        return lax.rem(b, NSLOT)

    def fetch_block(b, wait):
        m = slot(b)
        j0 = window(b)
        cg = pltpu.async_copy(g_hbm.at[:, pl.ds(j0, JB)],
                              blk.at[m, pl.ds(0, G_DIM), :], bsem.at[m])
        cr = pltpu.async_copy(r_hbm.at[:, pl.ds(j0, JB)],
                              blk.at[m, pl.ds(G_DIM, R_DIM), :], bsem.at[m])
        if wait:
            cg.wait()
            cr.wait()

    def drain_block(b):
        m = slot(b)
        pltpu.make_async_copy(
            g_hbm.at[:, pl.ds(0, JB)], blk.at[m], bsem.at[m]).wait()

    def drain_scatter(p):
        pltpu.make_async_copy(
            out_hbm.at[pl.ds(0, GROUP), :], comb.at[p], ssem.at[p]).wait()

    def extract(cb, rowi, gp, colv):
        mv = jnp.full((16,), slot(cb), jnp.int32)
        for gi in range(0, G_DIM, 16):
            comb[gp, rowi, pl.ds(gi, 16)] = plsc.load_gather(
                blk, [mv, lane + gi, colv])
        for gi in range(0, R_DIM, 16):
            comb[gp, rowi, pl.ds(G_DIM + gi, 16)] = plsc.load_gather(
                blk, [mv, G_DIM + lane + gi, colv])

    pltpu.sync_copy(sidx_hbm.at[wid], sidx_v.at[pl.ds(0, B_PER_W)])
    pltpu.sync_copy(spos_hbm.at[wid], spos_v)

    b0 = sidx_v[pl.ds(0, 16)][0] >> JB_SHIFT
    fetch_block(b0, True)
    for k in range(1, NSLOT):
        fetch_block(b0 + k, False)

    def chunk_body(ci, cur_b):
        sl = ci * 16
        g = ci >> 2
        gp = g & 1
        rowbase = sl & (GROUP - 1)
        jv = sidx_v[pl.ds(sl, 16)]
        b_last = (jv >> JB_SHIFT)[15]

        @pl.when(((ci & 3) == 0) & (g >= 2))
        def _():
            drain_scatter(gp)

        def fast(cb):
            colsc[:] = jv - window(cb)
            for h in range(16):
                colv = plsc.load_gather(colsc, [hsplat[h]])
                extract(cb, rowbase + h, gp, colv)
            return cb

        def slow(cb):
            def hit_body(h, cbi):
                j = sidx_v[pl.ds(sl + h, 16)][0]
                b = j >> JB_SHIFT
                reload = b != cbi
                jump = b != cbi + 1

                @pl.when(reload & jnp.logical_not(jump))
                def _():
                    drain_block(b)
                    fetch_block(b + NSLOT - 1, False)

                @pl.when(reload & jump)
                def _():
                    for k in range(1, NSLOT):
                        drain_block(cbi + k)
                    fetch_block(b, True)
                    for k in range(1, NSLOT):
                        fetch_block(b + k, False)

                colv = jnp.full((16,), j - window(b), jnp.int32)
                extract(b, rowbase + h, gp, colv)
                return b

            return lax.fori_loop(0, 16, hit_body, cb)

        new_b = lax.cond(b_last == cur_b, fast, slow, cur_b)

        @pl.when((ci & 3) == 3)
        def _():
            pltpu.async_copy(
                comb.at[gp], out_hbm.at[spos_v.at[g]], ssem.at[gp])

        return new_b

    last_b = lax.fori_loop(0, N_CHUNKS, chunk_body, b0)

    for k in range(1, NSLOT):
        drain_block(last_b + k)
    drain_scatter(0)
    drain_scatter(1)


def _emb_call(sidx2, spos3, glove_t, rand_t):
    kern = pl.kernel(
        _emb_body,
        out_type=jax.ShapeDtypeStruct((B_TOTAL, OUT_DIM), jnp.float32),
        mesh=plsc.VectorSubcoreMesh(core_axis_name="c", subcore_axis_name="s"),
        compiler_params=pltpu.CompilerParams(needs_layout_passes=False),
        scratch_types=[
            pltpu.VMEM((B_PER_W + 16,), jnp.int32),
            pltpu.VMEM((N_GROUPS, GROUP), jnp.int32),
            pltpu.VMEM((NSLOT, OUT_DIM, JB), jnp.float32),
            pltpu.VMEM((2, GROUP, OUT_DIM), jnp.float32),
            pltpu.VMEM((16,), jnp.int32),
            pltpu.SemaphoreType.DMA((NSLOT,)),
            pltpu.SemaphoreType.DMA((2,)),
        ],
    )
    return kern(sidx2, spos3, glove_t, rand_t)


def kernel(x, glove_weight, rand_weight):
    xt_flat = x.T.reshape(B_TOTAL)
    k = lax.iota(jnp.int32, B_TOTAL)
    pos = (k % BATCH) * NB_WORDS + k // BATCH
    sidx, spos = lax.sort([xt_flat, pos], num_keys=1, is_stable=False)
    sidx2 = sidx.reshape(NW, B_PER_W)
    spos3 = spos.reshape(NW, N_GROUPS, GROUP)
    out = _emb_call(sidx2, spos3, glove_weight.T, rand_weight.T)
    return out.reshape(BATCH, NB_WORDS, OUT_DIM)
